# Initial kernel scaffold; baseline (speedup 1.0000x reference)
import jax
import jax.numpy as jnp
from jax.experimental import pallas as pl


def kernel(id, value, weight, bias):
    raise NotImplementedError("write your pallas kernel here")


# speedup vs baseline: 1.1161x; 1.1161x over previous
import functools

import jax
import jax.numpy as jnp
from jax import lax
from jax.experimental import pallas as pl
from jax.experimental.pallas import tpu as pltpu
from jax.experimental.pallas import tpu_sc as plsc

B = 16384
F = 26
NC = 2
NS = 16
NW = NC * NS
CHUNK = (B * F) // NW


def _sc_gather(ids_flat, table):
    idw = ids_flat.reshape(NW, CHUNK)
    mesh = plsc.VectorSubcoreMesh(core_axis_name="c", subcore_axis_name="s")

    @functools.partial(
        pl.kernel,
        out_type=jax.ShapeDtypeStruct((B * F,), jnp.float32),
        mesh=mesh,
        scratch_types=[
            pltpu.VMEM((CHUNK,), jnp.int32),
            pltpu.VMEM((CHUNK,), jnp.float32),
            pltpu.SemaphoreType.DMA,
        ],
    )
    def body(idw_hbm, table_hbm, out_hbm, idx_v, g_v, sem):
        w = lax.axis_index("s") * NC + lax.axis_index("c")
        pltpu.sync_copy(idw_hbm.at[w], idx_v)
        pltpu.async_copy(table_hbm.at[idx_v], g_v, sem).wait()
        pltpu.sync_copy(g_v, out_hbm.at[pl.ds(w * CHUNK, CHUNK)])

    return body(idw, table)


def _tc_reduce(gathered, value, bias):
    blk = 2048

    def body(b_ref, g_ref, v_ref, o_ref):
        o_ref[...] = jnp.sum(g_ref[...] * v_ref[...], axis=1) + b_ref[0]

    return pl.pallas_call(
        body,
        grid=(B // blk,),
        in_specs=[
            pl.BlockSpec(memory_space=pltpu.SMEM),
            pl.BlockSpec((blk, F), lambda i: (i, 0)),
            pl.BlockSpec((blk, F), lambda i: (i, 0)),
        ],
        out_specs=pl.BlockSpec((blk,), lambda i: (i,)),
        out_shape=jax.ShapeDtypeStruct((B,), jnp.float32),
    )(bias, gathered, value)


def kernel(id, value, weight, bias):
    ids_flat = id.astype(jnp.int32).reshape(-1)
    table = weight.reshape(-1)
    gathered = _sc_gather(ids_flat, table).reshape(B, F)
    return _tc_reduce(gathered, value, bias)


# speedup vs baseline: 1.1687x; 1.0471x over previous
import functools

import jax
import jax.numpy as jnp
from jax import lax
from jax.experimental import pallas as pl
from jax.experimental.pallas import tpu as pltpu
from jax.experimental.pallas import tpu_sc as plsc

B = 16384
F = 26
NC = 2
NS = 16
NW = NC * NS
BPW = B // NW
CHUNK = BPW * F
L = 16


def _sc_linear(ids_flat, vals_flat, table):
    mesh = plsc.VectorSubcoreMesh(core_axis_name="c", subcore_axis_name="s")

    @functools.partial(
        pl.kernel,
        out_type=jax.ShapeDtypeStruct((B,), jnp.float32),
        mesh=mesh,
        compiler_params=pltpu.CompilerParams(needs_layout_passes=False),
        scratch_types=[
            pltpu.VMEM((CHUNK,), jnp.int32),
            pltpu.VMEM((CHUNK,), jnp.int32),
            pltpu.VMEM((CHUNK,), jnp.float32),
            pltpu.VMEM((CHUNK,), jnp.float32),
            pltpu.VMEM((BPW,), jnp.float32),
            pltpu.SemaphoreType.DMA,
            pltpu.SemaphoreType.DMA,
        ],
    )
    def body(ids_hbm, vals_hbm, table_hbm, out_hbm,
             idr_v, idf_v, g_v, val_v, out_v, gsem, vsem):
        w = lax.axis_index("s") * NC + lax.axis_index("c")
        vcp = pltpu.async_copy(
            vals_hbm.at[pl.ds(w * CHUNK, CHUNK)], val_v, vsem)
        pltpu.sync_copy(ids_hbm.at[pl.ds(w * CHUNK, CHUNK)], idr_v)

        lanes = lax.iota(jnp.int32, L)

        def build(c, carry):
            f = c // (BPW // L)
            j0 = L * lax.rem(c, BPW // L)
            perm = (j0 + lanes) * F + f
            idf_v[pl.ds(c * L, L)] = plsc.load_gather(idr_v, [perm])
            return carry

        lax.fori_loop(0, CHUNK // L, build, 0)
        pltpu.async_copy(table_hbm.at[idf_v], g_v, gsem).wait()
        vcp.wait()

        def fma(jb, carry):
            base = jb * L
            vidx = (base + lanes) * F
            acc = jnp.zeros((L,), jnp.float32)
            for f in range(F):
                gchunk = g_v[pl.ds(f * BPW + base, L)]
                vchunk = plsc.load_gather(val_v, [vidx + f])
                acc = acc + gchunk * vchunk
            out_v[pl.ds(base, L)] = acc
            return carry

        lax.fori_loop(0, BPW // L, fma, 0)
        pltpu.sync_copy(out_v, out_hbm.at[pl.ds(w * BPW, BPW)])

    return body(ids_flat, vals_flat, table)


def kernel(id, value, weight, bias):
    ids_flat = id.astype(jnp.int32).reshape(-1)
    vals_flat = value.reshape(-1)
    table = weight.reshape(-1)
    return _sc_linear(ids_flat, vals_flat, table) + bias
